# baseline (speedup 1.0000x reference)
import functools

import jax
import jax.numpy as jnp
from jax import lax
from jax.experimental import pallas as pl
from jax.experimental.pallas import tpu as pltpu
from jax.experimental.pallas import tpu_sc as plsc

D = 32
NC, NS = 2, 16
NW = NC * NS
S = 4
G = 3
C0 = 104


def _emb_body(bpw, l, words_hbm, tags_hbm, wt_hbm, tt_hbm, out_hbm,
              widx, tidx, wrows, trows, sem_g, sem_w):
    wid = lax.axis_index("s") * NC + lax.axis_index("c")
    b0 = wid * bpw
    c1 = l - C0

    pltpu.sync_copy(words_hbm.at[pl.ds(b0, bpw)], widx)
    pltpu.sync_copy(tags_hbm.at[pl.ds(b0, bpw)], tidx)

    def gathers(b, r, start):
        for idx, rows, tab in ((widx, wrows, wt_hbm), (tidx, trows, tt_hbm)):
            for o, c in ((0, C0), (C0, c1)):
                cp = pltpu.make_async_copy(tab.at[idx.at[r, pl.ds(o, c)]],
                                           rows.at[b, pl.ds(o, c)],
                                           sem_g.at[b])
                cp.start() if start else cp.wait()

    def writes(b, r, start):
        for rows, o in ((wrows, 0), (trows, D)):
            cp = pltpu.make_async_copy(rows.at[b],
                                       out_hbm.at[b0 + r, :, pl.ds(o, D)],
                                       sem_w.at[b])
            cp.start() if start else cp.wait()

    @pl.loop(0, bpw // S)
    def _(g):
        for u in range(S):
            r = g * S + u

            @pl.when(g > 0)
            def _():
                writes(u, r - S, False)

            gathers(u, r, True)

            up = (u - G) % S

            @pl.when(r >= G)
            def _():
                gathers(up, r - G, False)
                writes(up, r - G, True)

    for t in range(G):
        r = bpw - G + t
        gathers(r % S, r, False)
        writes(r % S, r, True)
    for u in range(S):
        writes(u, bpw - S + ((u - (bpw - S) % S) % S), False)


def _build(nb, l):
    assert nb % NW == 0 and (nb // NW) % S == 0 and C0 % 8 == 0 and l > C0
    bpw = nb // NW
    mesh = plsc.VectorSubcoreMesh(core_axis_name="c", subcore_axis_name="s")
    return functools.partial(
        pl.kernel,
        out_type=jax.ShapeDtypeStruct((nb, l, 2 * D), jnp.float32),
        mesh=mesh,
        compiler_params=pltpu.CompilerParams(use_tc_tiling_on_sc=False),
        scratch_types=[
            pltpu.VMEM((bpw, l), jnp.int32),
            pltpu.VMEM((bpw, l), jnp.int32),
            pltpu.VMEM((S, l, D), jnp.float32),
            pltpu.VMEM((S, l, D), jnp.float32),
            pltpu.SemaphoreType.DMA((S,)),
            pltpu.SemaphoreType.DMA((S,)),
        ],
    )(functools.partial(_emb_body, bpw, l))


def kernel(words, tags, word_table, tag_table):
    nb, l = words.shape
    return _build(nb, l)(words, tags, word_table, tag_table)
